# Initial kernel scaffold; baseline (speedup 1.0000x reference)
import functools

import jax
import jax.numpy as jnp
from jax import lax
from jax.experimental import pallas as pl
from jax.experimental.pallas import tpu as pltpu
from jax.experimental.pallas import tpu_sc as plsc

_NN = 100000
_NE = 6400000
_NC, _NS, _L = 2, 16, 16
_NW = _NC * _NS
_NP = 102400
_SLICE = _NP // _NS
_CH = 4096
_NCH = 49
_EW = _CH * _NCH
_EPAD = _NW * _EW

_mesh = plsc.VectorSubcoreMesh(core_axis_name="c", subcore_axis_name="s")


@functools.partial(
    pl.kernel,
    out_type=jax.ShapeDtypeStruct((_NC, _NP), jnp.float32),
    mesh=_mesh,
    scratch_types=[
        pltpu.VMEM((128,), jnp.float32),
        pltpu.VMEM((8, _L), jnp.float32),
        pltpu.VMEM((_NP,), jnp.float32),
        pltpu.VMEM((_CH,), jnp.float32),
        pltpu.VMEM((_CH,), jnp.int32),
        pltpu.VMEM((_CH,), jnp.int32),
        pltpu.VMEM((_CH,), jnp.float32),
        pltpu.VMEM_SHARED((_NP,), jnp.float32),
    ],
)
def _zbl_sc(tab_hbm, cons_hbm, zq_hbm, di_hbm, ii_hbm, ij_hbm, zeros_hbm,
            part_hbm, tab_v, cons_v, zpn_v, di_v, ii_v, ij_v, vij_v, acc_sh):
    cid = lax.axis_index("c")
    sid = lax.axis_index("s")
    wid = sid * _NC + cid

    pltpu.sync_copy(tab_hbm, tab_v)
    pltpu.sync_copy(cons_hbm, cons_v)
    pltpu.sync_copy(zeros_hbm.at[pl.ds(sid * _SLICE, _SLICE)],
                    acc_sh.at[pl.ds(sid * _SLICE, _SLICE)])

    def zchunk(k, _):
        pltpu.sync_copy(zq_hbm.at[pl.ds(k * _CH, _CH)], ii_v)

        def zvec(v, _):
            z = ii_v[pl.ds(v * _L, _L)]
            zpn_v[pl.ds(k * _CH + v * _L, _L)] = plsc.load_gather(tab_v, [z])
            return 0

        lax.fori_loop(0, _CH // _L, zvec, 0)
        return 0

    lax.fori_loop(0, _NP // _CH, zchunk, 0)
    plsc.subcore_barrier()

    c0 = cons_v[0]
    c1 = cons_v[1]
    c2 = cons_v[2]
    c3 = cons_v[3]
    na0 = cons_v[4]
    na1 = cons_v[5]
    na2 = cons_v[6]
    na3 = cons_v[7]

    def echunk(k, _):
        base = wid * _EW + k * _CH
        pltpu.sync_copy(di_hbm.at[pl.ds(base, _CH)], di_v)
        pltpu.sync_copy(ii_hbm.at[pl.ds(base, _CH)], ii_v)
        pltpu.sync_copy(ij_hbm.at[pl.ds(base, _CH)], ij_v)

        def evec(v, _):
            sl = pl.ds(v * _L, _L)
            si = plsc.load_gather(zpn_v, [ii_v[sl]])
            sj = plsc.load_gather(zpn_v, [ij_v[sl]])
            t = di_v[sl] * (si + sj)
            acc = c0 * jnp.exp(na0 * t)
            acc = acc + c1 * jnp.exp(na1 * t)
            acc = acc + c2 * jnp.exp(na2 * t)
            acc = acc + c3 * jnp.exp(na3 * t)
            vij_v[sl] = acc
            return 0

        lax.fori_loop(0, _CH // _L, evec, 0)
        pltpu.sync_copy(vij_v, acc_sh.at[ii_v], add=True)
        return 0

    lax.fori_loop(0, _NCH, echunk, 0)
    plsc.subcore_barrier()
    pltpu.sync_copy(acc_sh.at[pl.ds(sid * _SLICE, _SLICE)],
                    part_hbm.at[cid, pl.ds(sid * _SLICE, _SLICE)])


def _combine_body(p_ref, o_ref):
    o_ref[...] = p_ref[0] + p_ref[1]


_combine = pl.pallas_call(
    _combine_body,
    out_shape=jax.ShapeDtypeStruct((_NP,), jnp.float32),
)


def kernel(Z, Dij, idx_i, idx_j, p, d, c, a):
    f32 = jnp.float32
    zf = jnp.arange(128, dtype=f32)
    tab = (zf ** p).astype(f32) / d
    cons = jnp.concatenate(
        [jnp.broadcast_to(c.astype(f32)[:, None], (4, _L)),
         jnp.broadcast_to(-a.astype(f32)[:, None], (4, _L))], axis=0)
    zq = jnp.zeros((_NP,), jnp.int32).at[:_NN].set(Z.astype(jnp.int32))
    pad = _EPAD - _NE
    di = jnp.concatenate([Dij.astype(f32), jnp.ones((pad,), f32)])
    ii = jnp.concatenate([idx_i.astype(jnp.int32),
                          jnp.full((pad,), _NP - 1, jnp.int32)])
    ij = jnp.concatenate([idx_j.astype(jnp.int32), jnp.zeros((pad,), jnp.int32)])
    zeros = jnp.zeros((_NP,), f32)
    part = _zbl_sc(tab, cons, zq, di, ii, ij, zeros)
    return _combine(part)[:_NN]


# speedup vs baseline: 190.4453x; 190.4453x over previous
import functools

import jax
import jax.numpy as jnp
from jax import lax
from jax.experimental import pallas as pl
from jax.experimental.pallas import tpu as pltpu
from jax.experimental.pallas import tpu_sc as plsc

_NN = 100000
_NE = 6400000
_NC, _NS, _L = 2, 16, 16
_NW = _NC * _NS
_NP = 102400
_SLICE = _NP // _NS
_CH = 4096
_NCH = 49
_EW = _CH * _NCH
_EPAD = _NW * _EW

_mesh = plsc.VectorSubcoreMesh(core_axis_name="c", subcore_axis_name="s")


@functools.partial(
    pl.kernel,
    out_type=jax.ShapeDtypeStruct((_NC, _NP), jnp.float32),
    mesh=_mesh,
    compiler_params=pltpu.CompilerParams(needs_layout_passes=False),
    scratch_types=[
        pltpu.VMEM((128,), jnp.float32),
        pltpu.VMEM((8, _L), jnp.float32),
        pltpu.VMEM((_NP,), jnp.float32),
        pltpu.VMEM((_CH,), jnp.float32),
        pltpu.VMEM((_CH,), jnp.int32),
        pltpu.VMEM((_CH,), jnp.int32),
        pltpu.VMEM((_CH,), jnp.float32),
        pltpu.VMEM_SHARED((_NP,), jnp.float32),
    ],
)
def _zbl_sc(tab_hbm, cons_hbm, zq_hbm, di_hbm, ii_hbm, ij_hbm, zeros_hbm,
            part_hbm, tab_v, cons_v, zpn_v, di_v, ii_v, ij_v, vij_v, acc_sh):
    cid = lax.axis_index("c")
    sid = lax.axis_index("s")
    wid = sid * _NC + cid

    pltpu.sync_copy(tab_hbm, tab_v)
    pltpu.sync_copy(cons_hbm, cons_v)
    pltpu.sync_copy(zeros_hbm.at[pl.ds(sid * _SLICE, _SLICE)],
                    acc_sh.at[pl.ds(sid * _SLICE, _SLICE)])

    def zchunk(k, _):
        pltpu.sync_copy(zq_hbm.at[pl.ds(k * _CH, _CH)], ii_v)

        def zvec(v, _):
            z = ii_v[pl.ds(v * _L, _L)]
            zpn_v[pl.ds(k * _CH + v * _L, _L)] = plsc.load_gather(tab_v, [z])
            return 0

        lax.fori_loop(0, _CH // _L, zvec, 0)
        return 0

    lax.fori_loop(0, _NP // _CH, zchunk, 0)
    plsc.subcore_barrier()

    c0 = cons_v[0]
    c1 = cons_v[1]
    c2 = cons_v[2]
    c3 = cons_v[3]
    na0 = cons_v[4]
    na1 = cons_v[5]
    na2 = cons_v[6]
    na3 = cons_v[7]

    def echunk(k, _):
        base = wid * _EW + k * _CH
        pltpu.sync_copy(di_hbm.at[pl.ds(base, _CH)], di_v)
        pltpu.sync_copy(ii_hbm.at[pl.ds(base, _CH)], ii_v)
        pltpu.sync_copy(ij_hbm.at[pl.ds(base, _CH)], ij_v)

        def evec(v, _):
            sl = pl.ds(v * _L, _L)
            si = plsc.load_gather(zpn_v, [ii_v[sl]])
            sj = plsc.load_gather(zpn_v, [ij_v[sl]])
            t = di_v[sl] * (si + sj)
            acc = c0 * jnp.exp(na0 * t)
            acc = acc + c1 * jnp.exp(na1 * t)
            acc = acc + c2 * jnp.exp(na2 * t)
            acc = acc + c3 * jnp.exp(na3 * t)
            vij_v[sl] = acc
            return 0

        lax.fori_loop(0, _CH // _L, evec, 0)
        pltpu.sync_copy(vij_v, acc_sh.at[ii_v], add=True)
        return 0

    lax.fori_loop(0, _NCH, echunk, 0)
    plsc.subcore_barrier()
    pltpu.sync_copy(acc_sh.at[pl.ds(sid * _SLICE, _SLICE)],
                    part_hbm.at[cid, pl.ds(sid * _SLICE, _SLICE)])


def _combine_body(p_ref, o_ref):
    o_ref[...] = p_ref[0] + p_ref[1]


_combine = pl.pallas_call(
    _combine_body,
    out_shape=jax.ShapeDtypeStruct((_NP,), jnp.float32),
)


def kernel(Z, Dij, idx_i, idx_j, p, d, c, a):
    f32 = jnp.float32
    zf = jnp.arange(128, dtype=f32)
    tab = (zf ** p).astype(f32) / d
    cons = jnp.concatenate(
        [jnp.broadcast_to(c.astype(f32)[:, None], (4, _L)),
         jnp.broadcast_to(-a.astype(f32)[:, None], (4, _L))], axis=0)
    zq = jnp.zeros((_NP,), jnp.int32).at[:_NN].set(Z.astype(jnp.int32))
    pad = _EPAD - _NE
    di = jnp.concatenate([Dij.astype(f32), jnp.ones((pad,), f32)])
    ii = jnp.concatenate([idx_i.astype(jnp.int32),
                          jnp.full((pad,), _NP - 1, jnp.int32)])
    ij = jnp.concatenate([idx_j.astype(jnp.int32), jnp.zeros((pad,), jnp.int32)])
    zeros = jnp.zeros((_NP,), f32)
    part = _zbl_sc(tab, cons, zq, di, ii, ij, zeros)
    return _combine(part)[:_NN]


# speedup vs baseline: 369.4613x; 1.9400x over previous
import functools

import jax
import jax.numpy as jnp
from jax import lax
from jax.experimental import pallas as pl
from jax.experimental.pallas import tpu as pltpu
from jax.experimental.pallas import tpu_sc as plsc

_NN = 100000
_NE = 6400000
_NC, _NS, _L = 2, 16, 16
_NW = _NC * _NS
_NP = 100352
_SLICE = _NP // _NS
_CH = 2048
_NCH = 98
_EW = _CH * _NCH
_EPAD = _NW * _EW
_EALLOC = _EPAD + 2 * _CH

_mesh = plsc.VectorSubcoreMesh(core_axis_name="c", subcore_axis_name="s")


@functools.partial(
    pl.kernel,
    out_type=jax.ShapeDtypeStruct((_NC, _NP), jnp.float32),
    mesh=_mesh,
    compiler_params=pltpu.CompilerParams(needs_layout_passes=False),
    scratch_types=[
        pltpu.VMEM((128,), jnp.float32),
        pltpu.VMEM((8, _L), jnp.float32),
        pltpu.VMEM((_NP,), jnp.float32),
        pltpu.VMEM((_CH,), jnp.float32),
        pltpu.VMEM((_CH,), jnp.float32),
        pltpu.VMEM((_CH,), jnp.int32),
        pltpu.VMEM((_CH,), jnp.int32),
        pltpu.VMEM((_CH,), jnp.int32),
        pltpu.VMEM((_CH,), jnp.int32),
        pltpu.VMEM((_CH,), jnp.float32),
        pltpu.VMEM((_CH,), jnp.float32),
        pltpu.VMEM_SHARED((_NP,), jnp.float32),
        pltpu.SemaphoreType.DMA,
        pltpu.SemaphoreType.DMA,
    ],
)
def _zbl_sc(tab_hbm, cons_hbm, zqf_hbm, di_hbm, ii_hbm, ij_hbm, zeros_hbm,
            part_hbm, tab_v, cons_v, zpn_v, di0_v, di1_v, ii0_v, ii1_v,
            ij0_v, ij1_v, vij0_v, vij1_v, acc_sh, sem0, sem1):
    cid = lax.axis_index("c")
    sid = lax.axis_index("s")
    wid = sid * _NC + cid
    sems = (sem0, sem1)
    dis = (di0_v, di1_v)
    iis = (ii0_v, ii1_v)
    ijs = (ij0_v, ij1_v)
    vijs = (vij0_v, vij1_v)

    pltpu.sync_copy(tab_hbm, tab_v)
    pltpu.sync_copy(cons_hbm, cons_v)
    pltpu.sync_copy(zeros_hbm.at[pl.ds(sid * _SLICE, _SLICE)],
                    acc_sh.at[pl.ds(sid * _SLICE, _SLICE)])

    pltpu.sync_copy(zqf_hbm, zpn_v)

    @plsc.parallel_loop(0, _NP, step=_L, unroll=4)
    def _zbuild(i):
        z = plsc.bitcast(zpn_v[pl.ds(i, _L)], jnp.int32)
        zpn_v[pl.ds(i, _L)] = plsc.load_gather(tab_v, [z])

    plsc.subcore_barrier()

    c0 = cons_v[0]
    c1 = cons_v[1]
    c2 = cons_v[2]
    c3 = cons_v[3]
    na0 = cons_v[4]
    na1 = cons_v[5]
    na2 = cons_v[6]
    na3 = cons_v[7]

    def _fire(k, b):
        base = wid * _EW + k * _CH
        pltpu.async_copy(di_hbm.at[pl.ds(base, _CH)], dis[b], sems[b])
        pltpu.async_copy(ii_hbm.at[pl.ds(base, _CH)], iis[b], sems[b])
        pltpu.async_copy(ij_hbm.at[pl.ds(base, _CH)], ijs[b], sems[b])

    def _wait(b):
        pltpu.make_async_copy(di_hbm.at[pl.ds(0, _CH)], dis[b], sems[b]).wait()
        pltpu.make_async_copy(ii_hbm.at[pl.ds(0, _CH)], iis[b], sems[b]).wait()
        pltpu.make_async_copy(ij_hbm.at[pl.ds(0, _CH)], ijs[b], sems[b]).wait()

    _fire(0, 0)
    _fire(1, 1)

    def _pair(g, _):
        for b in range(2):
            k = 2 * g + b
            _wait(b)
            dib, iib, ijb, vb = dis[b], iis[b], ijs[b], vijs[b]

            @plsc.parallel_loop(0, _CH, step=_L, unroll=4)
            def _evec(i):
                sl = pl.ds(i, _L)
                si = plsc.load_gather(zpn_v, [iib[sl]])
                sj = plsc.load_gather(zpn_v, [ijb[sl]])
                t = dib[sl] * (si + sj)
                acc = c0 * jnp.exp(na0 * t)
                acc = acc + c1 * jnp.exp(na1 * t)
                acc = acc + c2 * jnp.exp(na2 * t)
                acc = acc + c3 * jnp.exp(na3 * t)
                vb[sl] = acc

            pltpu.sync_copy(vb, acc_sh.at[iib], add=True)
            _fire(k + 2, b)
        return 0

    lax.fori_loop(0, _NCH // 2, _pair, 0)
    _wait(0)
    _wait(1)

    plsc.subcore_barrier()
    pltpu.sync_copy(acc_sh.at[pl.ds(sid * _SLICE, _SLICE)],
                    part_hbm.at[cid, pl.ds(sid * _SLICE, _SLICE)])


def _combine_body(p_ref, o_ref):
    o_ref[...] = p_ref[0] + p_ref[1]


_combine = pl.pallas_call(
    _combine_body,
    out_shape=jax.ShapeDtypeStruct((_NP,), jnp.float32),
)


def kernel(Z, Dij, idx_i, idx_j, p, d, c, a):
    f32 = jnp.float32
    zf = jnp.arange(128, dtype=f32)
    tab = (zf ** p).astype(f32) / d
    cons = jnp.concatenate(
        [jnp.broadcast_to(c.astype(f32)[:, None], (4, _L)),
         jnp.broadcast_to(-a.astype(f32)[:, None], (4, _L))], axis=0)
    zq = jnp.zeros((_NP,), jnp.int32).at[:_NN].set(Z.astype(jnp.int32))
    zqf = lax.bitcast_convert_type(zq, f32)
    pad = _EALLOC - _NE
    di = jnp.concatenate([Dij.astype(f32), jnp.ones((pad,), f32)])
    ii = jnp.concatenate([idx_i.astype(jnp.int32),
                          jnp.full((pad,), _NP - 1, jnp.int32)])
    ij = jnp.concatenate([idx_j.astype(jnp.int32), jnp.zeros((pad,), jnp.int32)])
    zeros = jnp.zeros((_NP,), f32)
    part = _zbl_sc(tab, cons, zqf, di, ii, ij, zeros)
    return _combine(part)[:_NN]
